# Initial kernel scaffold; baseline (speedup 1.0000x reference)
import jax
import jax.numpy as jnp
from jax.experimental import pallas as pl


def kernel(x, edge_index, W1, b1, W2, b2):
    raise NotImplementedError("write your pallas kernel here")


# speedup vs baseline: 23.9302x; 23.9302x over previous
import functools
import jax
import jax.numpy as jnp
from jax import lax
from jax.experimental import pallas as pl
from jax.experimental.pallas import tpu as pltpu
from jax.experimental.pallas import tpu_sc as plsc

NC = 2
NS = 16
L = 16
NW = NC * NS
CH = 128
IB = 8


def _mesh():
    return plsc.VectorSubcoreMesh(
        core_axis_name="c", subcore_axis_name="s", num_cores=NC, num_subcores=NS
    )


def _sc_degree(dstp, ones_hbm, npad, kch):
    ept = npad // NS

    @functools.partial(
        pl.kernel,
        out_type=jax.ShapeDtypeStruct((NC, npad), jnp.float32),
        mesh=_mesh(),
        scratch_types=[
            pltpu.VMEM((IB, CH), jnp.int32),
            pltpu.VMEM((CH,), jnp.float32),
            pltpu.VMEM((ept,), jnp.float32),
            pltpu.VMEM_SHARED((npad,), jnp.float32),
        ],
    )
    def deg_kernel(dst_hbm, ones_hbm_ref, deg_out, idx_v, ones_v, buf_v, acc):
        cid = lax.axis_index("c")
        sid = lax.axis_index("s")
        wid = cid * NS + sid
        base = sid * ept
        pltpu.sync_copy(ones_hbm_ref.at[pl.ds(0, CH)], ones_v)
        pltpu.sync_copy(ones_hbm_ref.at[pl.ds(base, ept)], buf_v)
        pltpu.sync_copy(buf_v, acc.at[pl.ds(base, ept)])
        plsc.subcore_barrier()

        def body(ki, _):
            pltpu.sync_copy(dst_hbm.at[wid, pl.ds(ki * IB, IB)], idx_v)

            def inner(jj, carry):
                pltpu.sync_copy(ones_v, acc.at[idx_v.at[jj]], add=True)
                return carry

            return lax.fori_loop(0, IB, inner, _)

        lax.fori_loop(0, kch // IB, body, None)
        plsc.subcore_barrier()
        pltpu.sync_copy(acc.at[pl.ds(base, ept)], buf_v)
        pltpu.sync_copy(buf_v, deg_out.at[cid, pl.ds(base, ept)])

    return deg_kernel(dstp, ones_hbm)


def _sc_aggregate_rows(srcp, dstp, y, npad, kch):
    width = y.shape[1]
    rows_pt = npad // NS
    wchunk = 128

    @functools.partial(
        pl.kernel,
        out_type=jax.ShapeDtypeStruct((NC, npad, width), jnp.float32),
        mesh=_mesh(),
        scratch_types=[
            pltpu.VMEM((IB, CH), jnp.int32),
            pltpu.VMEM((IB, CH), jnp.int32),
            pltpu.VMEM((CH, width), jnp.float32),
            pltpu.VMEM_SHARED((npad, width), jnp.float32),
        ],
    )
    def agg_kernel(src_hbm, dst_hbm, y_hbm, z_out, src_v, dst_v, gbuf, acc):
        cid = lax.axis_index("c")
        sid = lax.axis_index("s")
        wid = cid * NS + sid
        base = sid * rows_pt

        def init_body(k, _):
            pltpu.sync_copy(y_hbm.at[pl.ds(base + k * wchunk, wchunk)], gbuf)
            pltpu.sync_copy(gbuf, acc.at[pl.ds(base + k * wchunk, wchunk)])
            return _

        lax.fori_loop(0, rows_pt // wchunk, init_body, None)
        plsc.subcore_barrier()

        def body(ki, _):
            pltpu.sync_copy(src_hbm.at[wid, pl.ds(ki * IB, IB)], src_v)
            pltpu.sync_copy(dst_hbm.at[wid, pl.ds(ki * IB, IB)], dst_v)

            def inner(jj, carry):
                pltpu.sync_copy(y_hbm.at[src_v.at[jj]], gbuf)
                pltpu.sync_copy(gbuf, acc.at[dst_v.at[jj]], add=True)
                return carry

            return lax.fori_loop(0, IB, inner, _)

        lax.fori_loop(0, kch // IB, body, None)
        plsc.subcore_barrier()

        def out_body(k, _):
            pltpu.sync_copy(acc.at[pl.ds(base + k * wchunk, wchunk)], gbuf)
            pltpu.sync_copy(gbuf, z_out.at[cid, pl.ds(base + k * wchunk, wchunk)])
            return _

        lax.fori_loop(0, rows_pt // wchunk, out_body, None)

    return agg_kernel(srcp, dstp, y)


def _sc_aggregate_elems(gidx, sidx, y2f, nelem, kch2):
    ept = nelem // NS

    @functools.partial(
        pl.kernel,
        out_type=jax.ShapeDtypeStruct((NC, nelem), jnp.float32),
        mesh=_mesh(),
        scratch_types=[
            pltpu.VMEM((IB, CH), jnp.int32),
            pltpu.VMEM((IB, CH), jnp.int32),
            pltpu.VMEM((CH,), jnp.float32),
            pltpu.VMEM((ept,), jnp.float32),
            pltpu.VMEM_SHARED((nelem,), jnp.float32),
            pltpu.VMEM_SHARED((nelem,), jnp.float32),
        ],
    )
    def agg2_kernel(g_hbm, s_hbm, y_hbm, z_out, g_v, s_v, gbuf, ybuf, ytab, acc):
        cid = lax.axis_index("c")
        sid = lax.axis_index("s")
        wid = cid * NS + sid
        base = sid * ept
        pltpu.sync_copy(y_hbm.at[pl.ds(base, ept)], ybuf)
        pltpu.sync_copy(ybuf, ytab.at[pl.ds(base, ept)])
        pltpu.sync_copy(ybuf, acc.at[pl.ds(base, ept)])
        plsc.subcore_barrier()

        def body(ki, _):
            pltpu.sync_copy(g_hbm.at[wid, pl.ds(ki * IB, IB)], g_v)
            pltpu.sync_copy(s_hbm.at[wid, pl.ds(ki * IB, IB)], s_v)

            def inner(jj, carry):
                pltpu.sync_copy(ytab.at[g_v.at[jj]], gbuf)
                pltpu.sync_copy(gbuf, acc.at[s_v.at[jj]], add=True)
                return carry

            return lax.fori_loop(0, IB, inner, _)

        lax.fori_loop(0, kch2 // IB, body, None)
        plsc.subcore_barrier()
        pltpu.sync_copy(acc.at[pl.ds(base, ept)], ybuf)
        pltpu.sync_copy(ybuf, z_out.at[cid, pl.ds(base, ept)])

    return agg2_kernel(gidx, sidx, y2f)


_BR = 1024


def _tc1(xp, W1, d0, d1):
    npad, d_in = xp.shape

    def body(x_ref, w_ref, d0_ref, d1_ref, y_ref):
        dinv = lax.rsqrt(d0_ref[...] + d1_ref[...] - 1.0)
        xw = jnp.dot(x_ref[...], w_ref[...], preferred_element_type=jnp.float32)
        y_ref[...] = xw * dinv

    return pl.pallas_call(
        body,
        grid=(npad // _BR,),
        in_specs=[
            pl.BlockSpec((_BR, d_in), lambda i: (i, 0)),
            pl.BlockSpec((d_in, d_in), lambda i: (0, 0)),
            pl.BlockSpec((_BR, 1), lambda i: (i, 0)),
            pl.BlockSpec((_BR, 1), lambda i: (i, 0)),
        ],
        out_specs=pl.BlockSpec((_BR, d_in), lambda i: (i, 0)),
        out_shape=jax.ShapeDtypeStruct((npad, d_in), jnp.float32),
    )(xp, W1, d0, d1)


def _tc2(z0, z1, y, d0, d1, W2p, b1row):
    npad, d_in = y.shape

    def body(z0_ref, z1_ref, y_ref, d0_ref, d1_ref, w_ref, b_ref, y2_ref):
        dinv = lax.rsqrt(d0_ref[...] + d1_ref[...] - 1.0)
        h = jnp.maximum(
            (z0_ref[...] + z1_ref[...] - y_ref[...]) * dinv + b_ref[...], 0.0
        )
        y2_ref[...] = (
            jnp.dot(h, w_ref[...], preferred_element_type=jnp.float32) * dinv
        )

    return pl.pallas_call(
        body,
        grid=(npad // _BR,),
        in_specs=[
            pl.BlockSpec((_BR, d_in), lambda i: (i, 0)),
            pl.BlockSpec((_BR, d_in), lambda i: (i, 0)),
            pl.BlockSpec((_BR, d_in), lambda i: (i, 0)),
            pl.BlockSpec((_BR, 1), lambda i: (i, 0)),
            pl.BlockSpec((_BR, 1), lambda i: (i, 0)),
            pl.BlockSpec((d_in, L), lambda i: (0, 0)),
            pl.BlockSpec((1, d_in), lambda i: (0, 0)),
        ],
        out_specs=pl.BlockSpec((_BR, L), lambda i: (i, 0)),
        out_shape=jax.ShapeDtypeStruct((npad, L), jnp.float32),
    )(z0, z1, y, d0, d1, W2p, b1row)


def _tc3(z20, z21, y2c, d0, d1, b2row):
    npad, dd = y2c.shape

    def body(z0_ref, z1_ref, y2_ref, d0_ref, d1_ref, b_ref, o_ref):
        dinv = lax.rsqrt(d0_ref[...] + d1_ref[...] - 1.0)
        o_ref[...] = (z0_ref[...] + z1_ref[...] - y2_ref[...]) * dinv + b_ref[...]

    return pl.pallas_call(
        body,
        grid=(npad // _BR,),
        in_specs=[
            pl.BlockSpec((_BR, dd), lambda i: (i, 0)),
            pl.BlockSpec((_BR, dd), lambda i: (i, 0)),
            pl.BlockSpec((_BR, dd), lambda i: (i, 0)),
            pl.BlockSpec((_BR, 1), lambda i: (i, 0)),
            pl.BlockSpec((_BR, 1), lambda i: (i, 0)),
            pl.BlockSpec((1, dd), lambda i: (0, 0)),
        ],
        out_specs=pl.BlockSpec((_BR, dd), lambda i: (i, 0)),
        out_shape=jax.ShapeDtypeStruct((npad, dd), jnp.float32),
    )(z20, z21, y2c, d0, d1, b2row)


def kernel(x, edge_index, W1, b1, W2, b2):
    n, d_in = x.shape
    e = edge_index.shape[1]
    d_out = W2.shape[1]
    npad = ((n + 2 * NS * 128 - 1) // (2 * NS * 128)) * (2 * NS * 128)
    jrows = npad - n

    epw = -(-e // NW)
    kch = -(-(-(-epw // CH)) // IB) * IB
    e_pad = NW * kch * CH - e

    xp = jnp.pad(x, ((0, jrows), (0, 0)))
    src = edge_index[0]
    dst = edge_index[1]
    padidx = (n + (jnp.arange(e_pad, dtype=jnp.int32) % jrows)).astype(jnp.int32)
    srcp = jnp.concatenate([src, padidx]).reshape(NW, kch, CH)
    dstp = jnp.concatenate([dst, padidx]).reshape(NW, kch, CH)
    ones1d = jnp.ones((npad,), jnp.float32)
    W2p = jnp.pad(W2, ((0, 0), (0, L - d_out)))
    b1row = b1.reshape(1, -1)
    b2row = b2.reshape(1, d_out)

    ecc = CH // d_out
    kch2 = -(-(-(-epw // ecc)) // IB) * IB
    e_pad2 = NW * kch2 * ecc - e
    padidx2 = (n + (jnp.arange(e_pad2, dtype=jnp.int32) % jrows)).astype(jnp.int32)
    src2 = jnp.concatenate([src, padidx2])
    dst2 = jnp.concatenate([dst, padidx2])
    col = jnp.arange(d_out, dtype=jnp.int32)[None, :]
    gidx = (src2[:, None] * d_out + col).reshape(NW, kch2, CH)
    sidx = (dst2[:, None] * d_out + col).reshape(NW, kch2, CH)

    deg = _sc_degree(dstp, ones1d, npad, kch)
    d0 = deg[0].reshape(npad, 1)
    d1 = deg[1].reshape(npad, 1)
    y = _tc1(xp, W1, d0, d1)
    z = _sc_aggregate_rows(srcp, dstp, y, npad, kch)
    y2 = _tc2(z[0], z[1], y, d0, d1, W2p, b1row)
    y2c = y2[:, :d_out]
    y2f = y2c.reshape(npad * d_out)
    z2 = _sc_aggregate_elems(gidx, sidx, y2f, npad * d_out, kch2)
    outp = _tc3(
        z2[0].reshape(npad, d_out), z2[1].reshape(npad, d_out), y2c, d0, d1, b2row
    )
    return outp[:n, :d_out]


# speedup vs baseline: 27.3194x; 1.1416x over previous
import functools
import jax
import jax.numpy as jnp
from jax import lax
from jax.experimental import pallas as pl
from jax.experimental.pallas import tpu as pltpu
from jax.experimental.pallas import tpu_sc as plsc

NC = 2
NS = 16
L = 16
NW = NC * NS
CHR = 64
CH2 = 128
GB = 4
IB = 8


def _mesh():
    return plsc.VectorSubcoreMesh(
        core_axis_name="c", subcore_axis_name="s", num_cores=NC, num_subcores=NS
    )


def _sc_degree(dstp, ones_hbm, npad, kch):
    ept = npad // NS

    @functools.partial(
        pl.kernel,
        out_type=jax.ShapeDtypeStruct((NC, npad), jnp.float32),
        mesh=_mesh(),
        scratch_types=[
            pltpu.VMEM((2, IB, CHR), jnp.int32),
            pltpu.VMEM((CHR,), jnp.float32),
            pltpu.VMEM((ept,), jnp.float32),
            pltpu.VMEM_SHARED((npad,), jnp.float32),
            pltpu.SemaphoreType.DMA,
        ],
    )
    def deg_kernel(dst_hbm, ones_hbm_ref, deg_out, idx_v, ones_v, buf_v, acc,
                   ssem):
        cid = lax.axis_index("c")
        sid = lax.axis_index("s")
        wid = cid * NS + sid
        base = sid * ept
        pltpu.sync_copy(ones_hbm_ref.at[pl.ds(0, CHR)], ones_v)
        pltpu.sync_copy(ones_hbm_ref.at[pl.ds(base, ept)], buf_v)
        pltpu.sync_copy(buf_v, acc.at[pl.ds(base, ept)])
        plsc.subcore_barrier()

        def body(b, _):
            p = lax.rem(b, 2)

            @pl.when(b >= 2)
            def _drain():
                def w(i, c):
                    pltpu.make_async_copy(
                        ones_v, acc.at[idx_v.at[p, 0]], ssem
                    ).wait()
                    return c
                lax.fori_loop(0, IB, w, None)

            pltpu.sync_copy(dst_hbm.at[wid, pl.ds(b * IB, IB)], idx_v.at[p])
            for u in range(IB):
                pltpu.async_copy(ones_v, acc.at[idx_v.at[p, u]], ssem, add=True)
            return _

        nblk = kch // IB
        lax.fori_loop(0, nblk, body, None)

        def wfin(i, c):
            pltpu.make_async_copy(ones_v, acc.at[idx_v.at[0, 0]], ssem).wait()
            return c

        lax.fori_loop(0, 2 * IB if nblk >= 2 else nblk * IB, wfin, None)
        plsc.subcore_barrier()
        pltpu.sync_copy(acc.at[pl.ds(base, ept)], buf_v)
        pltpu.sync_copy(buf_v, deg_out.at[cid, pl.ds(base, ept)])

    return deg_kernel(dstp, ones_hbm)


def _sc_aggregate_rows(srcp, dstp, y, npad, kch):
    width = y.shape[1]
    rows_pt = npad // NS
    wchunk = CHR

    @functools.partial(
        pl.kernel,
        out_type=jax.ShapeDtypeStruct((NC, npad, width), jnp.float32),
        mesh=_mesh(),
        scratch_types=[
            pltpu.VMEM((2, GB, CHR), jnp.int32),
            pltpu.VMEM((2, GB, CHR), jnp.int32),
            pltpu.VMEM((GB, CHR, width), jnp.float32),
            pltpu.VMEM_SHARED((npad, width), jnp.float32),
            [pltpu.SemaphoreType.DMA] * GB,
            [pltpu.SemaphoreType.DMA] * GB,
        ],
    )
    def agg_kernel(src_hbm, dst_hbm, y_hbm, z_out, src_v, dst_v, gbuf, acc,
                   gsems, ssems):
        cid = lax.axis_index("c")
        sid = lax.axis_index("s")
        wid = cid * NS + sid
        base = sid * rows_pt

        def init_body(k, _):
            pltpu.sync_copy(y_hbm.at[pl.ds(base + k * wchunk, wchunk)],
                            gbuf.at[0])
            pltpu.sync_copy(gbuf.at[0],
                            acc.at[pl.ds(base + k * wchunk, wchunk)])
            return _

        lax.fori_loop(0, rows_pt // wchunk, init_body, None)
        plsc.subcore_barrier()

        def body(g, _):
            p = lax.rem(g, 2)
            pltpu.sync_copy(src_hbm.at[wid, pl.ds(g * GB, GB)], src_v.at[p])
            pltpu.sync_copy(dst_hbm.at[wid, pl.ds(g * GB, GB)], dst_v.at[p])
            for u in range(GB):
                @pl.when(g >= 1)
                def _wait_prev():
                    pltpu.make_async_copy(
                        gbuf.at[u], acc.at[dst_v.at[p, u]], ssems[u]
                    ).wait()
                pltpu.async_copy(y_hbm.at[src_v.at[p, u]], gbuf.at[u], gsems[u])
            for u in range(GB):
                pltpu.make_async_copy(
                    y_hbm.at[src_v.at[p, u]], gbuf.at[u], gsems[u]
                ).wait()
                pltpu.async_copy(
                    gbuf.at[u], acc.at[dst_v.at[p, u]], ssems[u], add=True
                )
            return _

        lax.fori_loop(0, kch // GB, body, None)
        for u in range(GB):
            pltpu.make_async_copy(
                gbuf.at[u], acc.at[dst_v.at[0, u]], ssems[u]
            ).wait()
        plsc.subcore_barrier()

        def out_body(k, _):
            pltpu.sync_copy(acc.at[pl.ds(base + k * wchunk, wchunk)],
                            gbuf.at[0])
            pltpu.sync_copy(gbuf.at[0],
                            z_out.at[cid, pl.ds(base + k * wchunk, wchunk)])
            return _

        lax.fori_loop(0, rows_pt // wchunk, out_body, None)

    return agg_kernel(srcp, dstp, y)


def _sc_aggregate_elems(gidx, sidx, y2f, nelem, kch2):
    ept = nelem // NS

    @functools.partial(
        pl.kernel,
        out_type=jax.ShapeDtypeStruct((NC, nelem), jnp.float32),
        mesh=_mesh(),
        scratch_types=[
            pltpu.VMEM((2, GB, CH2), jnp.int32),
            pltpu.VMEM((2, GB, CH2), jnp.int32),
            pltpu.VMEM((GB, CH2), jnp.float32),
            pltpu.VMEM((ept,), jnp.float32),
            pltpu.VMEM_SHARED((nelem,), jnp.float32),
            pltpu.VMEM_SHARED((nelem,), jnp.float32),
            [pltpu.SemaphoreType.DMA] * GB,
            [pltpu.SemaphoreType.DMA] * GB,
        ],
    )
    def agg2_kernel(g_hbm, s_hbm, y_hbm, z_out, g_v, s_v, gbuf, ybuf, ytab, acc,
                    gsems, ssems):
        cid = lax.axis_index("c")
        sid = lax.axis_index("s")
        wid = cid * NS + sid
        base = sid * ept
        pltpu.sync_copy(y_hbm.at[pl.ds(base, ept)], ybuf)
        pltpu.sync_copy(ybuf, ytab.at[pl.ds(base, ept)])
        pltpu.sync_copy(ybuf, acc.at[pl.ds(base, ept)])
        plsc.subcore_barrier()

        def body(g, _):
            p = lax.rem(g, 2)
            pltpu.sync_copy(g_hbm.at[wid, pl.ds(g * GB, GB)], g_v.at[p])
            pltpu.sync_copy(s_hbm.at[wid, pl.ds(g * GB, GB)], s_v.at[p])
            for u in range(GB):
                @pl.when(g >= 1)
                def _wait_prev():
                    pltpu.make_async_copy(
                        gbuf.at[u], acc.at[s_v.at[p, u]], ssems[u]
                    ).wait()
                pltpu.async_copy(ytab.at[g_v.at[p, u]], gbuf.at[u], gsems[u])
            for u in range(GB):
                pltpu.make_async_copy(
                    ytab.at[g_v.at[p, u]], gbuf.at[u], gsems[u]
                ).wait()
                pltpu.async_copy(
                    gbuf.at[u], acc.at[s_v.at[p, u]], ssems[u], add=True
                )
            return _

        lax.fori_loop(0, kch2 // GB, body, None)
        for u in range(GB):
            pltpu.make_async_copy(
                gbuf.at[u], acc.at[s_v.at[0, u]], ssems[u]
            ).wait()
        plsc.subcore_barrier()
        pltpu.sync_copy(acc.at[pl.ds(base, ept)], ybuf)
        pltpu.sync_copy(ybuf, z_out.at[cid, pl.ds(base, ept)])

    return agg2_kernel(gidx, sidx, y2f)


_BR = 1024


def _tc1(xp, W1, d0, d1):
    npad, d_in = xp.shape

    def body(x_ref, w_ref, d0_ref, d1_ref, y_ref):
        dinv = lax.rsqrt(d0_ref[...] + d1_ref[...] - 1.0)
        xw = jnp.dot(x_ref[...], w_ref[...], preferred_element_type=jnp.float32)
        y_ref[...] = xw * dinv

    return pl.pallas_call(
        body,
        grid=(npad // _BR,),
        in_specs=[
            pl.BlockSpec((_BR, d_in), lambda i: (i, 0)),
            pl.BlockSpec((d_in, d_in), lambda i: (0, 0)),
            pl.BlockSpec((_BR, 1), lambda i: (i, 0)),
            pl.BlockSpec((_BR, 1), lambda i: (i, 0)),
        ],
        out_specs=pl.BlockSpec((_BR, d_in), lambda i: (i, 0)),
        out_shape=jax.ShapeDtypeStruct((npad, d_in), jnp.float32),
    )(xp, W1, d0, d1)


def _tc2(z0, z1, y, d0, d1, W2p, b1row):
    npad, d_in = y.shape

    def body(z0_ref, z1_ref, y_ref, d0_ref, d1_ref, w_ref, b_ref, y2_ref):
        dinv = lax.rsqrt(d0_ref[...] + d1_ref[...] - 1.0)
        h = jnp.maximum(
            (z0_ref[...] + z1_ref[...] - y_ref[...]) * dinv + b_ref[...], 0.0
        )
        y2_ref[...] = (
            jnp.dot(h, w_ref[...], preferred_element_type=jnp.float32) * dinv
        )

    return pl.pallas_call(
        body,
        grid=(npad // _BR,),
        in_specs=[
            pl.BlockSpec((_BR, d_in), lambda i: (i, 0)),
            pl.BlockSpec((_BR, d_in), lambda i: (i, 0)),
            pl.BlockSpec((_BR, d_in), lambda i: (i, 0)),
            pl.BlockSpec((_BR, 1), lambda i: (i, 0)),
            pl.BlockSpec((_BR, 1), lambda i: (i, 0)),
            pl.BlockSpec((d_in, L), lambda i: (0, 0)),
            pl.BlockSpec((1, d_in), lambda i: (0, 0)),
        ],
        out_specs=pl.BlockSpec((_BR, L), lambda i: (i, 0)),
        out_shape=jax.ShapeDtypeStruct((npad, L), jnp.float32),
    )(z0, z1, y, d0, d1, W2p, b1row)


def _tc3(z20, z21, y2c, d0, d1, b2row):
    npad, dd = y2c.shape

    def body(z0_ref, z1_ref, y2_ref, d0_ref, d1_ref, b_ref, o_ref):
        dinv = lax.rsqrt(d0_ref[...] + d1_ref[...] - 1.0)
        o_ref[...] = (z0_ref[...] + z1_ref[...] - y2_ref[...]) * dinv + b_ref[...]

    return pl.pallas_call(
        body,
        grid=(npad // _BR,),
        in_specs=[
            pl.BlockSpec((_BR, dd), lambda i: (i, 0)),
            pl.BlockSpec((_BR, dd), lambda i: (i, 0)),
            pl.BlockSpec((_BR, dd), lambda i: (i, 0)),
            pl.BlockSpec((_BR, 1), lambda i: (i, 0)),
            pl.BlockSpec((_BR, 1), lambda i: (i, 0)),
            pl.BlockSpec((1, dd), lambda i: (0, 0)),
        ],
        out_specs=pl.BlockSpec((_BR, dd), lambda i: (i, 0)),
        out_shape=jax.ShapeDtypeStruct((npad, dd), jnp.float32),
    )(z20, z21, y2c, d0, d1, b2row)


def kernel(x, edge_index, W1, b1, W2, b2):
    n, d_in = x.shape
    e = edge_index.shape[1]
    d_out = W2.shape[1]
    align = NS * CHR * 2
    npad = ((n + align - 1) // align) * align
    jrows = npad - n

    epw = -(-e // NW)
    kch = -(-(-(-epw // CHR)) // IB) * IB
    e_pad = NW * kch * CHR - e

    xp = jnp.pad(x, ((0, jrows), (0, 0)))
    src = edge_index[0]
    dst = edge_index[1]
    padidx = (n + (jnp.arange(e_pad, dtype=jnp.int32) % jrows)).astype(jnp.int32)
    srcp = jnp.concatenate([src, padidx]).reshape(NW, kch, CHR)
    dstp = jnp.concatenate([dst, padidx]).reshape(NW, kch, CHR)
    ones1d = jnp.ones((npad,), jnp.float32)
    W2p = jnp.pad(W2, ((0, 0), (0, L - d_out)))
    b1row = b1.reshape(1, -1)
    b2row = b2.reshape(1, d_out)

    ecc = CH2 // d_out
    kch2 = -(-(-(-epw // ecc)) // GB) * GB
    e_pad2 = NW * kch2 * ecc - e
    padidx2 = (n + (jnp.arange(e_pad2, dtype=jnp.int32) % jrows)).astype(jnp.int32)
    src2 = jnp.concatenate([src, padidx2])
    dst2 = jnp.concatenate([dst, padidx2])
    col = jnp.arange(d_out, dtype=jnp.int32)[None, :]
    gidx = (src2[:, None] * d_out + col).reshape(NW, kch2, CH2)
    sidx = (dst2[:, None] * d_out + col).reshape(NW, kch2, CH2)

    deg = _sc_degree(dstp, ones1d, npad, kch)
    d0 = deg[0].reshape(npad, 1)
    d1 = deg[1].reshape(npad, 1)
    y = _tc1(xp, W1, d0, d1)
    z = _sc_aggregate_rows(srcp, dstp, y, npad, kch)
    y2 = _tc2(z[0], z[1], y, d0, d1, W2p, b1row)
    y2c = y2[:, :d_out]
    y2f = y2c.reshape(npad * d_out)
    z2 = _sc_aggregate_elems(gidx, sidx, y2f, npad * d_out, kch2)
    outp = _tc3(
        z2[0].reshape(npad, d_out), z2[1].reshape(npad, d_out), y2c, d0, d1, b2row
    )
    return outp[:n, :d_out]


# speedup vs baseline: 28.5525x; 1.0451x over previous
import functools
import jax
import jax.numpy as jnp
from jax import lax
from jax.experimental import pallas as pl
from jax.experimental.pallas import tpu as pltpu
from jax.experimental.pallas import tpu_sc as plsc

NC = 2
NS = 16
L = 16
NW = NC * NS
CHR = 128
CH2 = 128
GB = 2
GB2 = 4
IB = 8


def _mesh():
    return plsc.VectorSubcoreMesh(
        core_axis_name="c", subcore_axis_name="s", num_cores=NC, num_subcores=NS
    )


def _sc_degree(dstp, ones_hbm, npad, kch):
    ept = npad // NS

    @functools.partial(
        pl.kernel,
        out_type=jax.ShapeDtypeStruct((NC, npad), jnp.float32),
        mesh=_mesh(),
        scratch_types=[
            pltpu.VMEM((2, IB, CHR), jnp.int32),
            pltpu.VMEM((CHR,), jnp.float32),
            pltpu.VMEM((ept,), jnp.float32),
            pltpu.VMEM_SHARED((npad,), jnp.float32),
            pltpu.SemaphoreType.DMA,
        ],
    )
    def deg_kernel(dst_hbm, ones_hbm_ref, deg_out, idx_v, ones_v, buf_v, acc,
                   ssem):
        cid = lax.axis_index("c")
        sid = lax.axis_index("s")
        wid = cid * NS + sid
        base = sid * ept
        pltpu.sync_copy(ones_hbm_ref.at[pl.ds(0, CHR)], ones_v)
        pltpu.sync_copy(ones_hbm_ref.at[pl.ds(base, ept)], buf_v)
        pltpu.sync_copy(buf_v, acc.at[pl.ds(base, ept)])
        plsc.subcore_barrier()

        def body(b, _):
            p = lax.rem(b, 2)

            @pl.when(b >= 2)
            def _drain():
                def w(i, c):
                    pltpu.make_async_copy(
                        ones_v, acc.at[idx_v.at[p, 0]], ssem
                    ).wait()
                    return c
                lax.fori_loop(0, IB, w, None)

            pltpu.sync_copy(dst_hbm.at[wid, pl.ds(b * IB, IB)], idx_v.at[p])
            for u in range(IB):
                pltpu.async_copy(ones_v, acc.at[idx_v.at[p, u]], ssem, add=True)
            return _

        nblk = kch // IB
        lax.fori_loop(0, nblk, body, None)

        def wfin(i, c):
            pltpu.make_async_copy(ones_v, acc.at[idx_v.at[0, 0]], ssem).wait()
            return c

        lax.fori_loop(0, 2 * IB if nblk >= 2 else nblk * IB, wfin, None)
        plsc.subcore_barrier()
        pltpu.sync_copy(acc.at[pl.ds(base, ept)], buf_v)
        pltpu.sync_copy(buf_v, deg_out.at[cid, pl.ds(base, ept)])

    return deg_kernel(dstp, ones_hbm)


def _sc_aggregate_rows(srcp, dstp, y, npad, kch):
    width = y.shape[1]
    rows_pt = npad // NS
    wchunk = CHR

    @functools.partial(
        pl.kernel,
        out_type=jax.ShapeDtypeStruct((NC, npad, width), jnp.float32),
        mesh=_mesh(),
        scratch_types=[
            pltpu.VMEM((2, GB, CHR), jnp.int32),
            pltpu.VMEM((2, GB, CHR), jnp.int32),
            pltpu.VMEM((GB, CHR, width), jnp.float32),
            pltpu.VMEM_SHARED((npad, width), jnp.float32),
            [pltpu.SemaphoreType.DMA] * GB,
            [pltpu.SemaphoreType.DMA] * GB,
        ],
    )
    def agg_kernel(src_hbm, dst_hbm, y_hbm, z_out, src_v, dst_v, gbuf, acc,
                   gsems, ssems):
        cid = lax.axis_index("c")
        sid = lax.axis_index("s")
        wid = cid * NS + sid
        base = sid * rows_pt

        def init_body(k, _):
            pltpu.sync_copy(y_hbm.at[pl.ds(base + k * wchunk, wchunk)],
                            gbuf.at[0])
            pltpu.sync_copy(gbuf.at[0],
                            acc.at[pl.ds(base + k * wchunk, wchunk)])
            return _

        lax.fori_loop(0, rows_pt // wchunk, init_body, None)
        plsc.subcore_barrier()

        def body(g, _):
            p = lax.rem(g, 2)
            pltpu.sync_copy(src_hbm.at[wid, pl.ds(g * GB, GB)], src_v.at[p])
            pltpu.sync_copy(dst_hbm.at[wid, pl.ds(g * GB, GB)], dst_v.at[p])
            for u in range(GB):
                @pl.when(g >= 1)
                def _wait_prev():
                    pltpu.make_async_copy(
                        gbuf.at[u], acc.at[dst_v.at[p, u]], ssems[u]
                    ).wait()
                pltpu.async_copy(y_hbm.at[src_v.at[p, u]], gbuf.at[u], gsems[u])
            for u in range(GB):
                pltpu.make_async_copy(
                    y_hbm.at[src_v.at[p, u]], gbuf.at[u], gsems[u]
                ).wait()
                pltpu.async_copy(
                    gbuf.at[u], acc.at[dst_v.at[p, u]], ssems[u], add=True
                )
            return _

        lax.fori_loop(0, kch // GB, body, None)
        for u in range(GB):
            pltpu.make_async_copy(
                gbuf.at[u], acc.at[dst_v.at[0, u]], ssems[u]
            ).wait()
        plsc.subcore_barrier()

        def out_body(k, _):
            pltpu.sync_copy(acc.at[pl.ds(base + k * wchunk, wchunk)],
                            gbuf.at[0])
            pltpu.sync_copy(gbuf.at[0],
                            z_out.at[cid, pl.ds(base + k * wchunk, wchunk)])
            return _

        lax.fori_loop(0, rows_pt // wchunk, out_body, None)

    return agg_kernel(srcp, dstp, y)


def _sc_aggregate_elems(gidx, sidx, y2f, nelem, kch2):
    ept = nelem // NS

    @functools.partial(
        pl.kernel,
        out_type=jax.ShapeDtypeStruct((NC, nelem), jnp.float32),
        mesh=_mesh(),
        scratch_types=[
            pltpu.VMEM((2, GB2, CH2), jnp.int32),
            pltpu.VMEM((2, GB2, CH2), jnp.int32),
            pltpu.VMEM((GB2, CH2), jnp.float32),
            pltpu.VMEM((ept,), jnp.float32),
            pltpu.VMEM_SHARED((nelem,), jnp.float32),
            pltpu.VMEM_SHARED((nelem,), jnp.float32),
            [pltpu.SemaphoreType.DMA] * GB2,
            [pltpu.SemaphoreType.DMA] * GB2,
        ],
    )
    def agg2_kernel(g_hbm, s_hbm, y_hbm, z_out, g_v, s_v, gbuf, ybuf, ytab, acc,
                    gsems, ssems):
        cid = lax.axis_index("c")
        sid = lax.axis_index("s")
        wid = cid * NS + sid
        base = sid * ept
        pltpu.sync_copy(y_hbm.at[pl.ds(base, ept)], ybuf)
        pltpu.sync_copy(ybuf, ytab.at[pl.ds(base, ept)])
        pltpu.sync_copy(ybuf, acc.at[pl.ds(base, ept)])
        plsc.subcore_barrier()

        def body(g, _):
            p = lax.rem(g, 2)
            pltpu.sync_copy(g_hbm.at[wid, pl.ds(g * GB2, GB2)], g_v.at[p])
            pltpu.sync_copy(s_hbm.at[wid, pl.ds(g * GB2, GB2)], s_v.at[p])
            for u in range(GB2):
                @pl.when(g >= 1)
                def _wait_prev():
                    pltpu.make_async_copy(
                        gbuf.at[u], acc.at[s_v.at[p, u]], ssems[u]
                    ).wait()
                pltpu.async_copy(ytab.at[g_v.at[p, u]], gbuf.at[u], gsems[u])
            for u in range(GB2):
                pltpu.make_async_copy(
                    ytab.at[g_v.at[p, u]], gbuf.at[u], gsems[u]
                ).wait()
                pltpu.async_copy(
                    gbuf.at[u], acc.at[s_v.at[p, u]], ssems[u], add=True
                )
            return _

        lax.fori_loop(0, kch2 // GB2, body, None)
        for u in range(GB2):
            pltpu.make_async_copy(
                gbuf.at[u], acc.at[s_v.at[0, u]], ssems[u]
            ).wait()
        plsc.subcore_barrier()
        pltpu.sync_copy(acc.at[pl.ds(base, ept)], ybuf)
        pltpu.sync_copy(ybuf, z_out.at[cid, pl.ds(base, ept)])

    return agg2_kernel(gidx, sidx, y2f)


_BR = 1024


def _tc1(xp, W1, d0, d1):
    npad, d_in = xp.shape

    def body(x_ref, w_ref, d0_ref, d1_ref, y_ref):
        dinv = lax.rsqrt(d0_ref[...] + d1_ref[...] - 1.0)
        xw = jnp.dot(x_ref[...], w_ref[...], preferred_element_type=jnp.float32)
        y_ref[...] = xw * dinv

    return pl.pallas_call(
        body,
        grid=(npad // _BR,),
        in_specs=[
            pl.BlockSpec((_BR, d_in), lambda i: (i, 0)),
            pl.BlockSpec((d_in, d_in), lambda i: (0, 0)),
            pl.BlockSpec((_BR, 1), lambda i: (i, 0)),
            pl.BlockSpec((_BR, 1), lambda i: (i, 0)),
        ],
        out_specs=pl.BlockSpec((_BR, d_in), lambda i: (i, 0)),
        out_shape=jax.ShapeDtypeStruct((npad, d_in), jnp.float32),
    )(xp, W1, d0, d1)


def _tc2(z0, z1, y, d0, d1, W2p, b1row):
    npad, d_in = y.shape

    def body(z0_ref, z1_ref, y_ref, d0_ref, d1_ref, w_ref, b_ref, y2_ref):
        dinv = lax.rsqrt(d0_ref[...] + d1_ref[...] - 1.0)
        h = jnp.maximum(
            (z0_ref[...] + z1_ref[...] - y_ref[...]) * dinv + b_ref[...], 0.0
        )
        y2_ref[...] = (
            jnp.dot(h, w_ref[...], preferred_element_type=jnp.float32) * dinv
        )

    return pl.pallas_call(
        body,
        grid=(npad // _BR,),
        in_specs=[
            pl.BlockSpec((_BR, d_in), lambda i: (i, 0)),
            pl.BlockSpec((_BR, d_in), lambda i: (i, 0)),
            pl.BlockSpec((_BR, d_in), lambda i: (i, 0)),
            pl.BlockSpec((_BR, 1), lambda i: (i, 0)),
            pl.BlockSpec((_BR, 1), lambda i: (i, 0)),
            pl.BlockSpec((d_in, L), lambda i: (0, 0)),
            pl.BlockSpec((1, d_in), lambda i: (0, 0)),
        ],
        out_specs=pl.BlockSpec((_BR, L), lambda i: (i, 0)),
        out_shape=jax.ShapeDtypeStruct((npad, L), jnp.float32),
    )(z0, z1, y, d0, d1, W2p, b1row)


def _tc3(z20, z21, y2c, d0, d1, b2row):
    npad, dd = y2c.shape

    def body(z0_ref, z1_ref, y2_ref, d0_ref, d1_ref, b_ref, o_ref):
        dinv = lax.rsqrt(d0_ref[...] + d1_ref[...] - 1.0)
        o_ref[...] = (z0_ref[...] + z1_ref[...] - y2_ref[...]) * dinv + b_ref[...]

    return pl.pallas_call(
        body,
        grid=(npad // _BR,),
        in_specs=[
            pl.BlockSpec((_BR, dd), lambda i: (i, 0)),
            pl.BlockSpec((_BR, dd), lambda i: (i, 0)),
            pl.BlockSpec((_BR, dd), lambda i: (i, 0)),
            pl.BlockSpec((_BR, 1), lambda i: (i, 0)),
            pl.BlockSpec((_BR, 1), lambda i: (i, 0)),
            pl.BlockSpec((1, dd), lambda i: (0, 0)),
        ],
        out_specs=pl.BlockSpec((_BR, dd), lambda i: (i, 0)),
        out_shape=jax.ShapeDtypeStruct((npad, dd), jnp.float32),
    )(z20, z21, y2c, d0, d1, b2row)


def kernel(x, edge_index, W1, b1, W2, b2):
    n, d_in = x.shape
    e = edge_index.shape[1]
    d_out = W2.shape[1]
    align = 2048
    npad = ((n + align - 1) // align) * align
    jrows = npad - n

    epw = -(-e // NW)
    kch = -(-(-(-epw // CHR)) // IB) * IB
    e_pad = NW * kch * CHR - e

    xp = jnp.pad(x, ((0, jrows), (0, 0)))
    src = edge_index[0]
    dst = edge_index[1]
    padidx = (n + (jnp.arange(e_pad, dtype=jnp.int32) % jrows)).astype(jnp.int32)
    srcp = jnp.concatenate([src, padidx]).reshape(NW, kch, CHR)
    dstp = jnp.concatenate([dst, padidx]).reshape(NW, kch, CHR)
    ones1d = jnp.ones((npad,), jnp.float32)
    W2p = jnp.pad(W2, ((0, 0), (0, L - d_out)))
    b1row = b1.reshape(1, -1)
    b2row = b2.reshape(1, d_out)

    ecc = CH2 // d_out
    kch2 = -(-(-(-epw // ecc)) // GB2) * GB2
    e_pad2 = NW * kch2 * ecc - e
    padidx2 = (n + (jnp.arange(e_pad2, dtype=jnp.int32) % jrows)).astype(jnp.int32)
    src2 = jnp.concatenate([src, padidx2])
    dst2 = jnp.concatenate([dst, padidx2])
    col = jnp.arange(d_out, dtype=jnp.int32)[None, :]
    gidx = (src2[:, None] * d_out + col).reshape(NW, kch2, CH2)
    sidx = (dst2[:, None] * d_out + col).reshape(NW, kch2, CH2)

    deg = _sc_degree(dstp, ones1d, npad, kch)
    d0 = deg[0].reshape(npad, 1)
    d1 = deg[1].reshape(npad, 1)
    y = _tc1(xp, W1, d0, d1)
    z = _sc_aggregate_rows(srcp, dstp, y, npad, kch)
    y2 = _tc2(z[0], z[1], y, d0, d1, W2p, b1row)
    y2c = y2[:, :d_out]
    y2f = y2c.reshape(npad * d_out)
    z2 = _sc_aggregate_elems(gidx, sidx, y2f, npad * d_out, kch2)
    outp = _tc3(
        z2[0].reshape(npad, d_out), z2[1].reshape(npad, d_out), y2c, d0, d1, b2row
    )
    return outp[:n, :d_out]


# speedup vs baseline: 29.6561x; 1.0387x over previous
import functools
import jax
import jax.numpy as jnp
from jax import lax
from jax.experimental import pallas as pl
from jax.experimental.pallas import tpu as pltpu
from jax.experimental.pallas import tpu_sc as plsc

NC = 2
NS = 16
L = 16
NW = NC * NS
CHR = 125
CH2 = 128
GB = 2
GB2 = 4
IB = 8


def _mesh():
    return plsc.VectorSubcoreMesh(
        core_axis_name="c", subcore_axis_name="s", num_cores=NC, num_subcores=NS
    )


def _sc_degree(dstp, ones_hbm, npad, kch):
    ept = npad // NS

    @functools.partial(
        pl.kernel,
        out_type=jax.ShapeDtypeStruct((NC, npad), jnp.float32),
        mesh=_mesh(),
        scratch_types=[
            pltpu.VMEM((2, IB, CHR), jnp.int32),
            pltpu.VMEM((CHR,), jnp.float32),
            pltpu.VMEM((ept,), jnp.float32),
            pltpu.VMEM_SHARED((npad,), jnp.float32),
            pltpu.SemaphoreType.DMA,
        ],
    )
    def deg_kernel(dst_hbm, ones_hbm_ref, deg_out, idx_v, ones_v, buf_v, acc,
                   ssem):
        cid = lax.axis_index("c")
        sid = lax.axis_index("s")
        wid = cid * NS + sid
        base = sid * ept
        pltpu.sync_copy(ones_hbm_ref.at[pl.ds(0, CHR)], ones_v)
        pltpu.sync_copy(ones_hbm_ref.at[pl.ds(base, ept)], buf_v)
        pltpu.sync_copy(buf_v, acc.at[pl.ds(base, ept)])
        plsc.subcore_barrier()

        def body(b, _):
            p = lax.rem(b, 2)

            @pl.when(b >= 2)
            def _drain():
                def w(i, c):
                    pltpu.make_async_copy(
                        ones_v, acc.at[idx_v.at[p, 0]], ssem
                    ).wait()
                    return c
                lax.fori_loop(0, IB, w, None)

            pltpu.sync_copy(dst_hbm.at[wid, pl.ds(b * IB, IB)], idx_v.at[p])
            for u in range(IB):
                pltpu.async_copy(ones_v, acc.at[idx_v.at[p, u]], ssem, add=True)
            return _

        nblk = kch // IB
        lax.fori_loop(0, nblk, body, None)

        def wfin(i, c):
            pltpu.make_async_copy(ones_v, acc.at[idx_v.at[0, 0]], ssem).wait()
            return c

        lax.fori_loop(0, 2 * IB if nblk >= 2 else nblk * IB, wfin, None)
        plsc.subcore_barrier()
        pltpu.sync_copy(acc.at[pl.ds(base, ept)], buf_v)
        pltpu.sync_copy(buf_v, deg_out.at[cid, pl.ds(base, ept)])

    return deg_kernel(dstp, ones_hbm)


def _sc_aggregate_rows(srcp, dstp, y, npad, kch):
    width = y.shape[1]
    rows_pt = npad // NS
    wchunk = 64

    @functools.partial(
        pl.kernel,
        out_type=jax.ShapeDtypeStruct((NC, npad, width), jnp.float32),
        mesh=_mesh(),
        scratch_types=[
            pltpu.VMEM((2, GB, CHR), jnp.int32),
            pltpu.VMEM((2, GB, CHR), jnp.int32),
            pltpu.VMEM((GB, CHR, width), jnp.float32),
            pltpu.VMEM_SHARED((npad, width), jnp.float32),
            [pltpu.SemaphoreType.DMA] * GB,
            [pltpu.SemaphoreType.DMA] * GB,
        ],
    )
    def agg_kernel(src_hbm, dst_hbm, y_hbm, z_out, src_v, dst_v, gbuf, acc,
                   gsems, ssems):
        cid = lax.axis_index("c")
        sid = lax.axis_index("s")
        wid = cid * NS + sid
        base = sid * rows_pt

        def init_body(k, _):
            pltpu.sync_copy(y_hbm.at[pl.ds(base + k * wchunk, wchunk)],
                            gbuf.at[0, pl.ds(0, wchunk)])
            pltpu.sync_copy(gbuf.at[0, pl.ds(0, wchunk)],
                            acc.at[pl.ds(base + k * wchunk, wchunk)])
            return _

        lax.fori_loop(0, rows_pt // wchunk, init_body, None)
        plsc.subcore_barrier()

        def body(g, _):
            p = lax.rem(g, 2)
            pltpu.sync_copy(src_hbm.at[wid, pl.ds(g * GB, GB)], src_v.at[p])
            pltpu.sync_copy(dst_hbm.at[wid, pl.ds(g * GB, GB)], dst_v.at[p])
            for u in range(GB):
                @pl.when(g >= 1)
                def _wait_prev():
                    pltpu.make_async_copy(
                        gbuf.at[u], acc.at[dst_v.at[p, u]], ssems[u]
                    ).wait()
                pltpu.async_copy(y_hbm.at[src_v.at[p, u]], gbuf.at[u], gsems[u])
            for u in range(GB):
                pltpu.make_async_copy(
                    y_hbm.at[src_v.at[p, u]], gbuf.at[u], gsems[u]
                ).wait()
                pltpu.async_copy(
                    gbuf.at[u], acc.at[dst_v.at[p, u]], ssems[u], add=True
                )
            return _

        lax.fori_loop(0, kch // GB, body, None)
        for u in range(GB):
            pltpu.make_async_copy(
                gbuf.at[u], acc.at[dst_v.at[0, u]], ssems[u]
            ).wait()
        plsc.subcore_barrier()

        def out_body(k, _):
            pltpu.sync_copy(acc.at[pl.ds(base + k * wchunk, wchunk)],
                            gbuf.at[0, pl.ds(0, wchunk)])
            pltpu.sync_copy(gbuf.at[0, pl.ds(0, wchunk)],
                            z_out.at[cid, pl.ds(base + k * wchunk, wchunk)])
            return _

        lax.fori_loop(0, rows_pt // wchunk, out_body, None)

    return agg_kernel(srcp, dstp, y)


def _sc_aggregate_elems(gidx, sidx, y2f, nelem, kch2):
    ept = nelem // NS

    @functools.partial(
        pl.kernel,
        out_type=jax.ShapeDtypeStruct((NC, nelem), jnp.float32),
        mesh=_mesh(),
        scratch_types=[
            pltpu.VMEM((2, GB2, CH2), jnp.int32),
            pltpu.VMEM((2, GB2, CH2), jnp.int32),
            pltpu.VMEM((GB2, CH2), jnp.float32),
            pltpu.VMEM((ept,), jnp.float32),
            pltpu.VMEM_SHARED((nelem,), jnp.float32),
            pltpu.VMEM_SHARED((nelem,), jnp.float32),
            [pltpu.SemaphoreType.DMA] * GB2,
            [pltpu.SemaphoreType.DMA] * GB2,
        ],
    )
    def agg2_kernel(g_hbm, s_hbm, y_hbm, z_out, g_v, s_v, gbuf, ybuf, ytab, acc,
                    gsems, ssems):
        cid = lax.axis_index("c")
        sid = lax.axis_index("s")
        wid = cid * NS + sid
        base = sid * ept
        pltpu.sync_copy(y_hbm.at[pl.ds(base, ept)], ybuf)
        pltpu.sync_copy(ybuf, ytab.at[pl.ds(base, ept)])
        pltpu.sync_copy(ybuf, acc.at[pl.ds(base, ept)])
        plsc.subcore_barrier()

        def body(g, _):
            p = lax.rem(g, 2)
            pltpu.sync_copy(g_hbm.at[wid, pl.ds(g * GB2, GB2)], g_v.at[p])
            pltpu.sync_copy(s_hbm.at[wid, pl.ds(g * GB2, GB2)], s_v.at[p])
            for u in range(GB2):
                @pl.when(g >= 1)
                def _wait_prev():
                    pltpu.make_async_copy(
                        gbuf.at[u], acc.at[s_v.at[p, u]], ssems[u]
                    ).wait()
                pltpu.async_copy(ytab.at[g_v.at[p, u]], gbuf.at[u], gsems[u])
            for u in range(GB2):
                pltpu.make_async_copy(
                    ytab.at[g_v.at[p, u]], gbuf.at[u], gsems[u]
                ).wait()
                pltpu.async_copy(
                    gbuf.at[u], acc.at[s_v.at[p, u]], ssems[u], add=True
                )
            return _

        lax.fori_loop(0, kch2 // GB2, body, None)
        for u in range(GB2):
            pltpu.make_async_copy(
                gbuf.at[u], acc.at[s_v.at[0, u]], ssems[u]
            ).wait()
        plsc.subcore_barrier()
        pltpu.sync_copy(acc.at[pl.ds(base, ept)], ybuf)
        pltpu.sync_copy(ybuf, z_out.at[cid, pl.ds(base, ept)])

    return agg2_kernel(gidx, sidx, y2f)


_BR = 1024


def _tc1(xp, W1, d0, d1):
    npad, d_in = xp.shape

    def body(x_ref, w_ref, d0_ref, d1_ref, y_ref):
        dinv = lax.rsqrt(d0_ref[...] + d1_ref[...] - 1.0)
        xw = jnp.dot(x_ref[...], w_ref[...], preferred_element_type=jnp.float32)
        y_ref[...] = xw * dinv

    return pl.pallas_call(
        body,
        grid=(npad // _BR,),
        in_specs=[
            pl.BlockSpec((_BR, d_in), lambda i: (i, 0)),
            pl.BlockSpec((d_in, d_in), lambda i: (0, 0)),
            pl.BlockSpec((_BR, 1), lambda i: (i, 0)),
            pl.BlockSpec((_BR, 1), lambda i: (i, 0)),
        ],
        out_specs=pl.BlockSpec((_BR, d_in), lambda i: (i, 0)),
        out_shape=jax.ShapeDtypeStruct((npad, d_in), jnp.float32),
    )(xp, W1, d0, d1)


def _tc2(z0, z1, y, d0, d1, W2p, b1row):
    npad, d_in = y.shape

    def body(z0_ref, z1_ref, y_ref, d0_ref, d1_ref, w_ref, b_ref, y2_ref):
        dinv = lax.rsqrt(d0_ref[...] + d1_ref[...] - 1.0)
        h = jnp.maximum(
            (z0_ref[...] + z1_ref[...] - y_ref[...]) * dinv + b_ref[...], 0.0
        )
        y2_ref[...] = (
            jnp.dot(h, w_ref[...], preferred_element_type=jnp.float32) * dinv
        )

    return pl.pallas_call(
        body,
        grid=(npad // _BR,),
        in_specs=[
            pl.BlockSpec((_BR, d_in), lambda i: (i, 0)),
            pl.BlockSpec((_BR, d_in), lambda i: (i, 0)),
            pl.BlockSpec((_BR, d_in), lambda i: (i, 0)),
            pl.BlockSpec((_BR, 1), lambda i: (i, 0)),
            pl.BlockSpec((_BR, 1), lambda i: (i, 0)),
            pl.BlockSpec((d_in, L), lambda i: (0, 0)),
            pl.BlockSpec((1, d_in), lambda i: (0, 0)),
        ],
        out_specs=pl.BlockSpec((_BR, L), lambda i: (i, 0)),
        out_shape=jax.ShapeDtypeStruct((npad, L), jnp.float32),
    )(z0, z1, y, d0, d1, W2p, b1row)


def _tc3(z20f, z21f, y2f, d0f, d1f, b2f):
    rows = z20f.shape[0] // 128

    def v(a):
        return a.reshape(rows, 128)

    def body(z0_ref, z1_ref, y2_ref, d0_ref, d1_ref, b_ref, o_ref):
        dinv = lax.rsqrt(d0_ref[...] + d1_ref[...] - 1.0)
        o_ref[...] = (z0_ref[...] + z1_ref[...] - y2_ref[...]) * dinv + b_ref[...]

    return pl.pallas_call(
        body,
        out_shape=jax.ShapeDtypeStruct((rows, 128), jnp.float32),
    )(v(z20f), v(z21f), v(y2f), v(d0f), v(d1f), v(b2f))


def kernel(x, edge_index, W1, b1, W2, b2):
    n, d_in = x.shape
    e = edge_index.shape[1]
    d_out = W2.shape[1]
    align = 2048
    npad = ((n + align - 1) // align) * align
    jrows = npad - n

    epw = -(-e // NW)
    if e % (NW * CHR * IB) == 0:
        kch = e // (NW * CHR)
        srcp = edge_index[0].reshape(NW, kch, CHR)
        dstp = edge_index[1].reshape(NW, kch, CHR)
    else:
        kch = -(-(-(-epw // CHR)) // IB) * IB
        e_pad = NW * kch * CHR - e
        padidx = (n + (jnp.arange(e_pad, dtype=jnp.int32) % jrows)).astype(
            jnp.int32)
        srcp = jnp.concatenate([edge_index[0], padidx]).reshape(NW, kch, CHR)
        dstp = jnp.concatenate([edge_index[1], padidx]).reshape(NW, kch, CHR)

    xp = jnp.pad(x, ((0, jrows), (0, 0)))
    src = edge_index[0]
    dst = edge_index[1]
    ones1d = jnp.ones((npad,), jnp.float32)
    W2p = jnp.pad(W2, ((0, 0), (0, L - d_out)))
    b1row = b1.reshape(1, -1)

    ecc = CH2 // d_out
    kch2 = -(-(-(-epw // ecc)) // GB2) * GB2
    e_pad2 = NW * kch2 * ecc - e
    padidx2 = (n + (jnp.arange(e_pad2, dtype=jnp.int32) % jrows)).astype(jnp.int32)
    src2 = jnp.concatenate([src, padidx2])
    dst2 = jnp.concatenate([dst, padidx2])
    col = jnp.arange(d_out, dtype=jnp.int32)[None, :]
    gidx = (src2[:, None] * d_out + col).reshape(NW, kch2, CH2)
    sidx = (dst2[:, None] * d_out + col).reshape(NW, kch2, CH2)

    deg = _sc_degree(dstp, ones1d, npad, kch)
    d0 = deg[0].reshape(npad, 1)
    d1 = deg[1].reshape(npad, 1)
    d0f = jnp.repeat(deg[0], d_out)
    d1f = jnp.repeat(deg[1], d_out)
    b2f = jnp.tile(b2, npad)
    y = _tc1(xp, W1, d0, d1)
    z = _sc_aggregate_rows(srcp, dstp, y, npad, kch)
    y2 = _tc2(z[0], z[1], y, d0, d1, W2p, b1row)
    y2f = y2[:, :d_out].reshape(npad * d_out)
    z2 = _sc_aggregate_elems(gidx, sidx, y2f, npad * d_out, kch2)
    outf = _tc3(z2[0], z2[1], y2f, d0f, d1f, b2f)
    return outf.reshape(npad, d_out)[:n]


# speedup vs baseline: 32.0098x; 1.0794x over previous
import functools
import jax
import jax.numpy as jnp
from jax import lax
from jax.experimental import pallas as pl
from jax.experimental.pallas import tpu as pltpu
from jax.experimental.pallas import tpu_sc as plsc

NC = 2
NS = 16
L = 16
NW = NC * NS
CHR = 125
CH2 = 128
GB = 2
GB2 = 4
IB = 8


def _mesh():
    return plsc.VectorSubcoreMesh(
        core_axis_name="c", subcore_axis_name="s", num_cores=NC, num_subcores=NS
    )


CC = 128


def _worker_range(wid, nch):
    bpw = nch // NW
    rem = nch % NW
    lo = wid * bpw + jnp.minimum(wid, rem)
    cnt = bpw + (wid < rem).astype(jnp.int32)
    return lo, cnt


def _sc_degree(ei, ones_hbm, npad, nch):
    ept = npad // NS

    @functools.partial(
        pl.kernel,
        out_type=jax.ShapeDtypeStruct((NC, npad), jnp.float32),
        mesh=_mesh(),
        scratch_types=[
            pltpu.VMEM((2, CC), jnp.int32),
            pltpu.VMEM((CC,), jnp.float32),
            pltpu.VMEM((ept,), jnp.float32),
            pltpu.VMEM_SHARED((npad,), jnp.float32),
            pltpu.SemaphoreType.DMA,
            pltpu.SemaphoreType.DMA,
        ],
    )
    def deg_kernel(ei_hbm, ones_hbm_ref, deg_out, idx_v, ones_v, buf_v, acc,
                   ssem0, ssem1):
        cid = lax.axis_index("c")
        sid = lax.axis_index("s")
        wid = cid * NS + sid
        base = sid * ept
        lo, cnt = _worker_range(wid, nch)
        ssems = [ssem0, ssem1]
        pltpu.sync_copy(ones_hbm_ref.at[pl.ds(0, CC)], ones_v)
        pltpu.sync_copy(ones_hbm_ref.at[pl.ds(base, ept)], buf_v)
        pltpu.sync_copy(buf_v, acc.at[pl.ds(base, ept)])
        plsc.subcore_barrier()

        def body(j, _):
            for b in range(2):
                @pl.when(lax.rem(j, 2) == b)
                def _step():
                    @pl.when(j >= 2)
                    def _drain():
                        pltpu.make_async_copy(
                            ones_v, acc.at[idx_v.at[b]], ssems[b]
                        ).wait()
                    pltpu.sync_copy(
                        ei_hbm.at[1, pl.ds((lo + j) * CC, CC)], idx_v.at[b]
                    )
                    pltpu.async_copy(
                        ones_v, acc.at[idx_v.at[b]], ssems[b], add=True
                    )
            return _

        lax.fori_loop(0, cnt, body, None)
        for b in range(2):
            @pl.when(cnt >= 1 + b)
            def _fin():
                pltpu.make_async_copy(
                    ones_v, acc.at[idx_v.at[b]], ssems[b]
                ).wait()
        plsc.subcore_barrier()
        pltpu.sync_copy(acc.at[pl.ds(base, ept)], buf_v)
        pltpu.sync_copy(buf_v, deg_out.at[cid, pl.ds(base, ept)])

    return deg_kernel(ei, ones_hbm)


def _sc_aggregate_rows(ei, y, npad, nch):
    width = y.shape[1]
    rows_pt = npad // NS
    wchunk = 64

    @functools.partial(
        pl.kernel,
        out_type=jax.ShapeDtypeStruct((NC, npad, width), jnp.float32),
        mesh=_mesh(),
        scratch_types=[
            pltpu.VMEM((2, 2, CC), jnp.int32),
            pltpu.VMEM((2, CC, width), jnp.float32),
            pltpu.VMEM_SHARED((npad, width), jnp.float32),
            pltpu.SemaphoreType.DMA,
            pltpu.SemaphoreType.DMA,
            pltpu.SemaphoreType.DMA,
            pltpu.SemaphoreType.DMA,
        ],
    )
    def agg_kernel(ei_hbm, y_hbm, z_out, exy_v, gbuf, acc,
                   gsem0, gsem1, ssem0, ssem1):
        cid = lax.axis_index("c")
        sid = lax.axis_index("s")
        wid = cid * NS + sid
        base = sid * rows_pt
        lo, cnt = _worker_range(wid, nch)
        gsems = [gsem0, gsem1]
        ssems = [ssem0, ssem1]

        def init_body(k, _):
            pltpu.sync_copy(y_hbm.at[pl.ds(base + k * wchunk, wchunk)],
                            gbuf.at[0, pl.ds(0, wchunk)])
            pltpu.sync_copy(gbuf.at[0, pl.ds(0, wchunk)],
                            acc.at[pl.ds(base + k * wchunk, wchunk)])
            return _

        lax.fori_loop(0, rows_pt // wchunk, init_body, None)
        plsc.subcore_barrier()

        def stage_and_gather(g, b):
            pltpu.sync_copy(
                ei_hbm.at[pl.ds(0, 2), pl.ds((lo + g) * CC, CC)], exy_v.at[b]
            )
            pltpu.async_copy(
                y_hbm.at[exy_v.at[b, 0]], gbuf.at[b], gsems[b]
            )

        @pl.when(cnt >= 1)
        def _pro():
            stage_and_gather(0, 0)

        def body(g, _):
            for b in range(2):
                nb = 1 - b

                @pl.when(lax.rem(g, 2) == b)
                def _step():
                    @pl.when(g + 1 < cnt)
                    def _ahead():
                        @pl.when(g >= 1)
                        def _drain():
                            pltpu.make_async_copy(
                                gbuf.at[nb], acc.at[exy_v.at[nb, 1]], ssems[nb]
                            ).wait()
                        stage_and_gather(g + 1, nb)
                    pltpu.make_async_copy(
                        y_hbm.at[exy_v.at[b, 0]], gbuf.at[b], gsems[b]
                    ).wait()
                    pltpu.async_copy(
                        gbuf.at[b], acc.at[exy_v.at[b, 1]], ssems[b], add=True
                    )
            return _

        lax.fori_loop(0, cnt, body, None)
        for b in range(2):
            @pl.when(cnt >= 1 + b)
            def _fin():
                pltpu.make_async_copy(
                    gbuf.at[b], acc.at[exy_v.at[b, 1]], ssems[b]
                ).wait()
        plsc.subcore_barrier()

        def out_body(k, _):
            pltpu.sync_copy(acc.at[pl.ds(base + k * wchunk, wchunk)],
                            gbuf.at[0, pl.ds(0, wchunk)])
            pltpu.sync_copy(gbuf.at[0, pl.ds(0, wchunk)],
                            z_out.at[cid, pl.ds(base + k * wchunk, wchunk)])
            return _

        lax.fori_loop(0, rows_pt // wchunk, out_body, None)

    return agg_kernel(ei, y)


def _sc_aggregate_elems(gidx, sidx, y2f, nelem, kch2):
    ept = nelem // NS

    @functools.partial(
        pl.kernel,
        out_type=jax.ShapeDtypeStruct((NC, nelem), jnp.float32),
        mesh=_mesh(),
        scratch_types=[
            pltpu.VMEM((2, GB2, CH2), jnp.int32),
            pltpu.VMEM((2, GB2, CH2), jnp.int32),
            pltpu.VMEM((GB2, CH2), jnp.float32),
            pltpu.VMEM((ept,), jnp.float32),
            pltpu.VMEM_SHARED((nelem,), jnp.float32),
            pltpu.VMEM_SHARED((nelem,), jnp.float32),
            [pltpu.SemaphoreType.DMA] * GB2,
            [pltpu.SemaphoreType.DMA] * GB2,
        ],
    )
    def agg2_kernel(g_hbm, s_hbm, y_hbm, z_out, g_v, s_v, gbuf, ybuf, ytab, acc,
                    gsems, ssems):
        cid = lax.axis_index("c")
        sid = lax.axis_index("s")
        wid = cid * NS + sid
        base = sid * ept
        pltpu.sync_copy(y_hbm.at[pl.ds(base, ept)], ybuf)
        pltpu.sync_copy(ybuf, ytab.at[pl.ds(base, ept)])
        pltpu.sync_copy(ybuf, acc.at[pl.ds(base, ept)])
        plsc.subcore_barrier()

        def body(g, _):
            p = lax.rem(g, 2)
            pltpu.sync_copy(g_hbm.at[wid, pl.ds(g * GB2, GB2)], g_v.at[p])
            pltpu.sync_copy(s_hbm.at[wid, pl.ds(g * GB2, GB2)], s_v.at[p])
            for u in range(GB2):
                @pl.when(g >= 1)
                def _wait_prev():
                    pltpu.make_async_copy(
                        gbuf.at[u], acc.at[s_v.at[p, u]], ssems[u]
                    ).wait()
                pltpu.async_copy(ytab.at[g_v.at[p, u]], gbuf.at[u], gsems[u])
            for u in range(GB2):
                pltpu.make_async_copy(
                    ytab.at[g_v.at[p, u]], gbuf.at[u], gsems[u]
                ).wait()
                pltpu.async_copy(
                    gbuf.at[u], acc.at[s_v.at[p, u]], ssems[u], add=True
                )
            return _

        lax.fori_loop(0, kch2 // GB2, body, None)
        for u in range(GB2):
            pltpu.make_async_copy(
                gbuf.at[u], acc.at[s_v.at[0, u]], ssems[u]
            ).wait()
        plsc.subcore_barrier()
        pltpu.sync_copy(acc.at[pl.ds(base, ept)], ybuf)
        pltpu.sync_copy(ybuf, z_out.at[cid, pl.ds(base, ept)])

    return agg2_kernel(gidx, sidx, y2f)


_BR = 1024


def _tc1(xp, W1, d0, d1):
    npad, d_in = xp.shape

    def body(x_ref, w_ref, d0_ref, d1_ref, y_ref):
        dinv = lax.rsqrt(d0_ref[...] + d1_ref[...] - 1.0)
        xw = jnp.dot(x_ref[...], w_ref[...], preferred_element_type=jnp.float32)
        y_ref[...] = xw * dinv

    return pl.pallas_call(
        body,
        grid=(npad // _BR,),
        in_specs=[
            pl.BlockSpec((_BR, d_in), lambda i: (i, 0)),
            pl.BlockSpec((d_in, d_in), lambda i: (0, 0)),
            pl.BlockSpec((_BR, 1), lambda i: (i, 0)),
            pl.BlockSpec((_BR, 1), lambda i: (i, 0)),
        ],
        out_specs=pl.BlockSpec((_BR, d_in), lambda i: (i, 0)),
        out_shape=jax.ShapeDtypeStruct((npad, d_in), jnp.float32),
    )(xp, W1, d0, d1)


def _tc2(z, y, d0, d1, W2p, b1row):
    npad, d_in = y.shape

    def body(z0_ref, z1_ref, y_ref, d0_ref, d1_ref, w_ref, b_ref, y2_ref):
        dinv = lax.rsqrt(d0_ref[...] + d1_ref[...] - 1.0)
        h = jnp.maximum(
            (z0_ref[0] + z1_ref[0] - y_ref[...]) * dinv + b_ref[...], 0.0
        )
        y2_ref[...] = (
            jnp.dot(h, w_ref[...], preferred_element_type=jnp.float32) * dinv
        )

    return pl.pallas_call(
        body,
        grid=(npad // _BR,),
        in_specs=[
            pl.BlockSpec((1, _BR, d_in), lambda i: (0, i, 0)),
            pl.BlockSpec((1, _BR, d_in), lambda i: (1, i, 0)),
            pl.BlockSpec((_BR, d_in), lambda i: (i, 0)),
            pl.BlockSpec((_BR, 1), lambda i: (i, 0)),
            pl.BlockSpec((_BR, 1), lambda i: (i, 0)),
            pl.BlockSpec((d_in, L), lambda i: (0, 0)),
            pl.BlockSpec((1, d_in), lambda i: (0, 0)),
        ],
        out_specs=pl.BlockSpec((_BR, L), lambda i: (i, 0)),
        out_shape=jax.ShapeDtypeStruct((npad, L), jnp.float32),
    )(z, z, y, d0, d1, W2p, b1row)


def _tc3(z20f, z21f, y2f, d0f, d1f, b2f):
    rows = z20f.shape[0] // 128

    def v(a):
        return a.reshape(rows, 128)

    def body(z0_ref, z1_ref, y2_ref, d0_ref, d1_ref, b_ref, o_ref):
        dinv = lax.rsqrt(d0_ref[...] + d1_ref[...] - 1.0)
        o_ref[...] = (z0_ref[...] + z1_ref[...] - y2_ref[...]) * dinv + b_ref[...]

    return pl.pallas_call(
        body,
        out_shape=jax.ShapeDtypeStruct((rows, 128), jnp.float32),
    )(v(z20f), v(z21f), v(y2f), v(d0f), v(d1f), v(b2f))


def kernel(x, edge_index, W1, b1, W2, b2):
    n, d_in = x.shape
    e = edge_index.shape[1]
    d_out = W2.shape[1]
    align = 2048
    npad = ((n + align - 1) // align) * align
    jrows = npad - n

    epw = -(-e // NW)
    if e % CC == 0:
        ei2 = edge_index
    else:
        e_pad = CC - e % CC
        padidx = (n + (jnp.arange(e_pad, dtype=jnp.int32) % jrows)).astype(
            jnp.int32)
        ei2 = jnp.concatenate(
            [edge_index, jnp.stack([padidx, padidx])], axis=1)
    nch = ei2.shape[1] // CC

    xp = jnp.pad(x, ((0, jrows), (0, 0)))
    src = edge_index[0]
    dst = edge_index[1]
    ones1d = jnp.ones((npad,), jnp.float32)
    W2p = jnp.pad(W2, ((0, 0), (0, L - d_out)))
    b1row = b1.reshape(1, -1)

    ecc = CH2 // d_out
    kch2 = -(-(-(-epw // ecc)) // GB2) * GB2
    e_pad2 = NW * kch2 * ecc - e
    padidx2 = (n + (jnp.arange(e_pad2, dtype=jnp.int32) % jrows)).astype(jnp.int32)
    src2 = jnp.concatenate([src, padidx2])
    dst2 = jnp.concatenate([dst, padidx2])
    col = jnp.arange(d_out, dtype=jnp.int32)[None, :]
    gidx = (src2[:, None] * d_out + col).reshape(NW, kch2, CH2)
    sidx = (dst2[:, None] * d_out + col).reshape(NW, kch2, CH2)

    deg = _sc_degree(ei2, ones1d, npad, nch)
    d0 = deg[0].reshape(npad, 1)
    d1 = deg[1].reshape(npad, 1)
    d0f = jnp.repeat(deg[0], d_out)
    d1f = jnp.repeat(deg[1], d_out)
    b2f = jnp.tile(b2, npad)
    y = _tc1(xp, W1, d0, d1)
    z = _sc_aggregate_rows(ei2, y, npad, nch)
    y2 = _tc2(z, y, d0, d1, W2p, b1row)
    y2f = y2[:, :d_out].reshape(npad * d_out)
    z2 = _sc_aggregate_elems(gidx, sidx, y2f, npad * d_out, kch2)
    outf = _tc3(z2[0], z2[1], y2f, d0f, d1f, b2f)
    return outf.reshape(npad, d_out)[:n]


# speedup vs baseline: 34.2870x; 1.0711x over previous
import functools
import jax
import jax.numpy as jnp
from jax import lax
from jax.experimental import pallas as pl
from jax.experimental.pallas import tpu as pltpu
from jax.experimental.pallas import tpu_sc as plsc

NC = 2
NS = 16
L = 16
NW = NC * NS
CHR = 125
CH2 = 128
GB = 2
GB2 = 8
IB = 8


def _mesh():
    return plsc.VectorSubcoreMesh(
        core_axis_name="c", subcore_axis_name="s", num_cores=NC, num_subcores=NS
    )


CC = 128


def _worker_range(wid, nch):
    bpw = nch // NW
    rem = nch % NW
    lo = wid * bpw + jnp.minimum(wid, rem)
    cnt = bpw + (wid < rem).astype(jnp.int32)
    return lo, cnt


def _sc_degree(ei, ones_hbm, npad, nch):
    ept = npad // NS

    @functools.partial(
        pl.kernel,
        out_type=jax.ShapeDtypeStruct((NC, npad), jnp.float32),
        mesh=_mesh(),
        scratch_types=[
            pltpu.VMEM((2, CC), jnp.int32),
            pltpu.VMEM((CC,), jnp.float32),
            pltpu.VMEM((ept,), jnp.float32),
            pltpu.VMEM_SHARED((npad,), jnp.float32),
            pltpu.SemaphoreType.DMA,
            pltpu.SemaphoreType.DMA,
        ],
    )
    def deg_kernel(ei_hbm, ones_hbm_ref, deg_out, idx_v, ones_v, buf_v, acc,
                   ssem0, ssem1):
        cid = lax.axis_index("c")
        sid = lax.axis_index("s")
        wid = cid * NS + sid
        base = sid * ept
        lo, cnt = _worker_range(wid, nch)
        ssems = [ssem0, ssem1]
        pltpu.sync_copy(ones_hbm_ref.at[pl.ds(0, CC)], ones_v)
        pltpu.sync_copy(ones_hbm_ref.at[pl.ds(base, ept)], buf_v)
        pltpu.sync_copy(buf_v, acc.at[pl.ds(base, ept)])
        plsc.subcore_barrier()

        def body(j, _):
            for b in range(2):
                @pl.when(lax.rem(j, 2) == b)
                def _step():
                    @pl.when(j >= 2)
                    def _drain():
                        pltpu.make_async_copy(
                            ones_v, acc.at[idx_v.at[b]], ssems[b]
                        ).wait()
                    pltpu.sync_copy(
                        ei_hbm.at[1, pl.ds((lo + j) * CC, CC)], idx_v.at[b]
                    )
                    pltpu.async_copy(
                        ones_v, acc.at[idx_v.at[b]], ssems[b], add=True
                    )
            return _

        lax.fori_loop(0, cnt, body, None)
        for b in range(2):
            @pl.when(cnt >= 1 + b)
            def _fin():
                pltpu.make_async_copy(
                    ones_v, acc.at[idx_v.at[b]], ssems[b]
                ).wait()
        plsc.subcore_barrier()
        pltpu.sync_copy(acc.at[pl.ds(base, ept)], buf_v)
        pltpu.sync_copy(buf_v, deg_out.at[cid, pl.ds(base, ept)])

    return deg_kernel(ei, ones_hbm)


def _sc_aggregate_rows(ei, y, npad, nch):
    width = y.shape[1]
    rows_pt = npad // NS
    wchunk = 64

    @functools.partial(
        pl.kernel,
        out_type=jax.ShapeDtypeStruct((NC, npad, width), jnp.float32),
        mesh=_mesh(),
        scratch_types=[
            pltpu.VMEM((2, 2, CC), jnp.int32),
            pltpu.VMEM((2, CC, width), jnp.float32),
            pltpu.VMEM_SHARED((npad, width), jnp.float32),
            pltpu.SemaphoreType.DMA,
            pltpu.SemaphoreType.DMA,
            pltpu.SemaphoreType.DMA,
            pltpu.SemaphoreType.DMA,
        ],
    )
    def agg_kernel(ei_hbm, y_hbm, z_out, exy_v, gbuf, acc,
                   gsem0, gsem1, ssem0, ssem1):
        cid = lax.axis_index("c")
        sid = lax.axis_index("s")
        wid = cid * NS + sid
        base = sid * rows_pt
        lo, cnt = _worker_range(wid, nch)
        gsems = [gsem0, gsem1]
        ssems = [ssem0, ssem1]

        def init_body(k, _):
            pltpu.sync_copy(y_hbm.at[pl.ds(base + k * wchunk, wchunk)],
                            gbuf.at[0, pl.ds(0, wchunk)])
            pltpu.sync_copy(gbuf.at[0, pl.ds(0, wchunk)],
                            acc.at[pl.ds(base + k * wchunk, wchunk)])
            return _

        lax.fori_loop(0, rows_pt // wchunk, init_body, None)
        plsc.subcore_barrier()

        def stage_and_gather(g, b):
            pltpu.sync_copy(
                ei_hbm.at[pl.ds(0, 2), pl.ds((lo + g) * CC, CC)], exy_v.at[b]
            )
            pltpu.async_copy(
                y_hbm.at[exy_v.at[b, 0]], gbuf.at[b], gsems[b]
            )

        @pl.when(cnt >= 1)
        def _pro():
            stage_and_gather(0, 0)

        def body(g, _):
            for b in range(2):
                nb = 1 - b

                @pl.when(lax.rem(g, 2) == b)
                def _step():
                    @pl.when(g + 1 < cnt)
                    def _ahead():
                        @pl.when(g >= 1)
                        def _drain():
                            pltpu.make_async_copy(
                                gbuf.at[nb], acc.at[exy_v.at[nb, 1]], ssems[nb]
                            ).wait()
                        stage_and_gather(g + 1, nb)
                    pltpu.make_async_copy(
                        y_hbm.at[exy_v.at[b, 0]], gbuf.at[b], gsems[b]
                    ).wait()
                    pltpu.async_copy(
                        gbuf.at[b], acc.at[exy_v.at[b, 1]], ssems[b], add=True
                    )
            return _

        lax.fori_loop(0, cnt, body, None)
        for b in range(2):
            @pl.when(cnt >= 1 + b)
            def _fin():
                pltpu.make_async_copy(
                    gbuf.at[b], acc.at[exy_v.at[b, 1]], ssems[b]
                ).wait()
        plsc.subcore_barrier()

        def out_body(k, _):
            pltpu.sync_copy(acc.at[pl.ds(base + k * wchunk, wchunk)],
                            gbuf.at[0, pl.ds(0, wchunk)])
            pltpu.sync_copy(gbuf.at[0, pl.ds(0, wchunk)],
                            z_out.at[cid, pl.ds(base + k * wchunk, wchunk)])
            return _

        lax.fori_loop(0, rows_pt // wchunk, out_body, None)

    return agg_kernel(ei, y)


def _sc_aggregate_elems(gidx, sidx, y2f, nelem, kch2):
    ept = nelem // NS

    @functools.partial(
        pl.kernel,
        out_type=jax.ShapeDtypeStruct((NC, nelem), jnp.float32),
        mesh=_mesh(),
        scratch_types=[
            pltpu.VMEM((2, GB2, CH2), jnp.int32),
            pltpu.VMEM((2, GB2, CH2), jnp.int32),
            pltpu.VMEM((GB2, CH2), jnp.float32),
            pltpu.VMEM((ept,), jnp.float32),
            pltpu.VMEM_SHARED((nelem,), jnp.float32),
            pltpu.VMEM_SHARED((nelem,), jnp.float32),
            [pltpu.SemaphoreType.DMA] * GB2,
            [pltpu.SemaphoreType.DMA] * GB2,
        ],
    )
    def agg2_kernel(g_hbm, s_hbm, y_hbm, z_out, g_v, s_v, gbuf, ybuf, ytab, acc,
                    gsems, ssems):
        cid = lax.axis_index("c")
        sid = lax.axis_index("s")
        wid = cid * NS + sid
        base = sid * ept
        pltpu.sync_copy(y_hbm.at[pl.ds(base, ept)], ybuf)
        pltpu.sync_copy(ybuf, ytab.at[pl.ds(base, ept)])
        pltpu.sync_copy(ybuf, acc.at[pl.ds(base, ept)])
        plsc.subcore_barrier()

        def body(g, _):
            p = lax.rem(g, 2)
            pltpu.sync_copy(g_hbm.at[wid, pl.ds(g * GB2, GB2)], g_v.at[p])
            pltpu.sync_copy(s_hbm.at[wid, pl.ds(g * GB2, GB2)], s_v.at[p])
            for u in range(GB2):
                @pl.when(g >= 1)
                def _wait_prev():
                    pltpu.make_async_copy(
                        gbuf.at[u], acc.at[s_v.at[p, u]], ssems[u]
                    ).wait()
                pltpu.async_copy(ytab.at[g_v.at[p, u]], gbuf.at[u], gsems[u])
            for u in range(GB2):
                pltpu.make_async_copy(
                    ytab.at[g_v.at[p, u]], gbuf.at[u], gsems[u]
                ).wait()
                pltpu.async_copy(
                    gbuf.at[u], acc.at[s_v.at[p, u]], ssems[u], add=True
                )
            return _

        lax.fori_loop(0, kch2 // GB2, body, None)
        for u in range(GB2):
            pltpu.make_async_copy(
                gbuf.at[u], acc.at[s_v.at[0, u]], ssems[u]
            ).wait()
        plsc.subcore_barrier()
        pltpu.sync_copy(acc.at[pl.ds(base, ept)], ybuf)
        pltpu.sync_copy(ybuf, z_out.at[cid, pl.ds(base, ept)])

    return agg2_kernel(gidx, sidx, y2f)


_BR = 1024


def _tc1(xp, W1, d0, d1):
    npad, d_in = xp.shape

    def body(x_ref, w_ref, d0_ref, d1_ref, y_ref):
        dinv = lax.rsqrt(d0_ref[...] + d1_ref[...] - 1.0)
        xw = jnp.dot(x_ref[...], w_ref[...], preferred_element_type=jnp.float32)
        y_ref[...] = xw * dinv

    return pl.pallas_call(
        body,
        grid=(npad // _BR,),
        in_specs=[
            pl.BlockSpec((_BR, d_in), lambda i: (i, 0)),
            pl.BlockSpec((d_in, d_in), lambda i: (0, 0)),
            pl.BlockSpec((_BR, 1), lambda i: (i, 0)),
            pl.BlockSpec((_BR, 1), lambda i: (i, 0)),
        ],
        out_specs=pl.BlockSpec((_BR, d_in), lambda i: (i, 0)),
        out_shape=jax.ShapeDtypeStruct((npad, d_in), jnp.float32),
    )(xp, W1, d0, d1)


def _tc2(z, y, d0, d1, W2p, b1row):
    npad, d_in = y.shape

    def body(z0_ref, z1_ref, y_ref, d0_ref, d1_ref, w_ref, b_ref, y2_ref):
        dinv = lax.rsqrt(d0_ref[...] + d1_ref[...] - 1.0)
        h = jnp.maximum(
            (z0_ref[0] + z1_ref[0] - y_ref[...]) * dinv + b_ref[...], 0.0
        )
        y2_ref[...] = (
            jnp.dot(h, w_ref[...], preferred_element_type=jnp.float32) * dinv
        )

    return pl.pallas_call(
        body,
        grid=(npad // _BR,),
        in_specs=[
            pl.BlockSpec((1, _BR, d_in), lambda i: (0, i, 0)),
            pl.BlockSpec((1, _BR, d_in), lambda i: (1, i, 0)),
            pl.BlockSpec((_BR, d_in), lambda i: (i, 0)),
            pl.BlockSpec((_BR, 1), lambda i: (i, 0)),
            pl.BlockSpec((_BR, 1), lambda i: (i, 0)),
            pl.BlockSpec((d_in, L), lambda i: (0, 0)),
            pl.BlockSpec((1, d_in), lambda i: (0, 0)),
        ],
        out_specs=pl.BlockSpec((_BR, L), lambda i: (i, 0)),
        out_shape=jax.ShapeDtypeStruct((npad, L), jnp.float32),
    )(z, z, y, d0, d1, W2p, b1row)


def _tc3(z20f, z21f, y2f, d0f, d1f, b2f):
    rows = z20f.shape[0] // 128

    def v(a):
        return a.reshape(rows, 128)

    def body(z0_ref, z1_ref, y2_ref, d0_ref, d1_ref, b_ref, o_ref):
        dinv = lax.rsqrt(d0_ref[...] + d1_ref[...] - 1.0)
        o_ref[...] = (z0_ref[...] + z1_ref[...] - y2_ref[...]) * dinv + b_ref[...]

    return pl.pallas_call(
        body,
        out_shape=jax.ShapeDtypeStruct((rows, 128), jnp.float32),
    )(v(z20f), v(z21f), v(y2f), v(d0f), v(d1f), v(b2f))


def kernel(x, edge_index, W1, b1, W2, b2):
    n, d_in = x.shape
    e = edge_index.shape[1]
    d_out = W2.shape[1]
    align = 2048
    npad = ((n + align - 1) // align) * align
    jrows = npad - n

    epw = -(-e // NW)
    if e % CC == 0:
        ei2 = edge_index
    else:
        e_pad = CC - e % CC
        padidx = (n + (jnp.arange(e_pad, dtype=jnp.int32) % jrows)).astype(
            jnp.int32)
        ei2 = jnp.concatenate(
            [edge_index, jnp.stack([padidx, padidx])], axis=1)
    nch = ei2.shape[1] // CC

    xp = jnp.pad(x, ((0, jrows), (0, 0)))
    src = edge_index[0]
    dst = edge_index[1]
    ones1d = jnp.ones((npad,), jnp.float32)
    W2p = jnp.pad(W2, ((0, 0), (0, L - d_out)))
    b1row = b1.reshape(1, -1)

    ecc = CH2 // d_out
    kch2 = -(-(-(-epw // ecc)) // GB2) * GB2
    e_pad2 = NW * kch2 * ecc - e
    padidx2 = (n + (jnp.arange(e_pad2, dtype=jnp.int32) % jrows)).astype(jnp.int32)
    src2 = jnp.concatenate([src, padidx2])
    dst2 = jnp.concatenate([dst, padidx2])
    col = jnp.arange(d_out, dtype=jnp.int32)[None, :]
    gidx = (src2[:, None] * d_out + col).reshape(NW, kch2, CH2)
    sidx = (dst2[:, None] * d_out + col).reshape(NW, kch2, CH2)

    deg = _sc_degree(ei2, ones1d, npad, nch)
    d0 = deg[0].reshape(npad, 1)
    d1 = deg[1].reshape(npad, 1)
    d0f = jnp.repeat(deg[0], d_out)
    d1f = jnp.repeat(deg[1], d_out)
    b2f = jnp.tile(b2, npad)
    y = _tc1(xp, W1, d0, d1)
    z = _sc_aggregate_rows(ei2, y, npad, nch)
    y2 = _tc2(z, y, d0, d1, W2p, b1row)
    y2f = y2[:, :d_out].reshape(npad * d_out)
    z2 = _sc_aggregate_elems(gidx, sidx, y2f, npad * d_out, kch2)
    outf = _tc3(z2[0], z2[1], y2f, d0f, d1f, b2f)
    return outf.reshape(npad, d_out)[:n]
